# baseline (speedup 1.0000x reference)
import functools

import jax
import jax.numpy as jnp
from jax import lax
from jax.experimental import pallas as pl
from jax.experimental.pallas import tpu as pltpu
from jax.experimental.pallas import tpu_sc as plsc

R, C = 8192, 4096
NUM_WORKERS = 32
ROWS_PER_WORKER = R // NUM_WORKERS
GROUP = 16
N_GROUPS = ROWS_PER_WORKER // GROUP

_mesh = plsc.VectorSubcoreMesh(core_axis_name="c", subcore_axis_name="s")


@functools.partial(
    pl.kernel,
    out_type=jax.ShapeDtypeStruct((R * C,), jnp.float32),
    mesh=_mesh,
    scratch_types=[pltpu.VMEM((GROUP * C,), jnp.float32)],
    compiler_params=pltpu.CompilerParams(needs_layout_passes=False),
)
def _cumsum_sc(x_hbm, out_hbm, buf):
    wid = lax.axis_index("s") * 2 + lax.axis_index("c")
    row0 = wid * ROWS_PER_WORKER
    base_idx = lax.iota(jnp.int32, 16) * C

    def group_body(g, _):
        r = row0 + g * GROUP
        pltpu.sync_copy(x_hbm.at[pl.ds(r * C, GROUP * C)], buf)

        def col_body(j, carry):
            s, idx = carry
            v = plsc.load_gather(buf, [idx])
            s = s + v
            plsc.store_scatter(buf, [idx], s)
            return (s, idx + 1)

        plsc.parallel_loop(
            0, C, 1, unroll=16,
            carry=(jnp.zeros((16,), jnp.float32), base_idx),
        )(col_body)
        pltpu.sync_copy(buf, out_hbm.at[pl.ds(r * C, GROUP * C)])
        return 0

    lax.fori_loop(0, N_GROUPS, group_body, 0)


def kernel(x):
    return _cumsum_sc(x.reshape(R * C)).reshape(R, C)


# speedup vs baseline: 1.3756x; 1.3756x over previous
import functools

import jax
import jax.numpy as jnp
from jax import lax
from jax.experimental import pallas as pl
from jax.experimental.pallas import tpu as pltpu
from jax.experimental.pallas import tpu_sc as plsc

R, C = 8192, 4096
NUM_WORKERS = 32
ROWS_PER_WORKER = R // NUM_WORKERS
GROUP = 16
N_GROUPS = ROWS_PER_WORKER // GROUP
C_CHUNK = 2048
N_CHUNKS = C // C_CHUNK

_mesh = plsc.VectorSubcoreMesh(core_axis_name="c", subcore_axis_name="s")


@functools.partial(
    pl.kernel,
    out_type=jax.ShapeDtypeStruct((R, C), jnp.float32),
    mesh=_mesh,
    scratch_types=[
        pltpu.VMEM((GROUP, C_CHUNK), jnp.float32),
        pltpu.VMEM((GROUP, C_CHUNK), jnp.float32),
    ],
    compiler_params=pltpu.CompilerParams(needs_layout_passes=False),
)
def _cumsum_sc(x_hbm, out_hbm, inb, outb):
    wid = lax.axis_index("s") * 2 + lax.axis_index("c")
    row0 = wid * ROWS_PER_WORKER
    lane = lax.iota(jnp.int32, 16)

    def group_body(g, _):
        r = row0 + g * GROUP
        s = jnp.zeros((16,), jnp.float32)
        for cc in range(N_CHUNKS):
            c0 = cc * C_CHUNK
            pltpu.sync_copy(x_hbm.at[pl.ds(r, GROUP), pl.ds(c0, C_CHUNK)], inb)

            def col_body(j, carry):
                s, col = carry
                v = plsc.load_gather(inb, [lane, col])
                s = s + v
                plsc.store_scatter(outb, [lane, col], s)
                return (s, col + 1)

            s, _ = plsc.parallel_loop(
                0, C_CHUNK, 1, unroll=8,
                carry=(s, jnp.zeros((16,), jnp.int32)),
            )(col_body)
            pltpu.sync_copy(outb, out_hbm.at[pl.ds(r, GROUP), pl.ds(c0, C_CHUNK)])
        return 0

    lax.fori_loop(0, N_GROUPS, group_body, 0)


def kernel(x):
    return _cumsum_sc(x)
